# baseline (speedup 1.0000x reference)
import dataclasses

import jax
import jax.numpy as jnp
from jax import lax
from jax.experimental import pallas as pl
from jax.experimental.pallas import tpu as pltpu
from jax.experimental.pallas import tpu_sc as plsc

N_NODES = 100000
N_SEG = 256
D = 128
CH = 80
NCHUNKS = N_NODES // CH
NC = 2
NS = 16
NW = NC * NS
CNT_W = 16
MULTI = 4
BIG = MULTI * CH
NBIG = 190
SC_ROWS = NBIG * BIG
T_STEPS = -(-NBIG // NW)
RB = 800
TC_BLOCKS = (N_NODES - SC_ROWS) // RB
TC_OFF = SC_ROWS // RB


def _sc_body(nodes_hbm, ids_hbm, zeros_hbm, zcnt_hbm, sums_hbm,
             counts_hbm, rows_v, ids_v, cnt_v, acc_sh,
             sem_r0, sem_r1, sem_i0, sem_i1):
  c = lax.axis_index("c")
  s = lax.axis_index("s")
  wid = c * NS + s
  sem_r = (sem_r0, sem_r1)
  sem_i = (sem_i0, sem_i1)

  pltpu.sync_copy(zcnt_hbm, cnt_v)

  @pl.when(s == 0)
  def _():
    pltpu.sync_copy(zeros_hbm, acc_sh)

  plsc.subcore_barrier()

  lane = lax.iota(jnp.int32, 16)
  one16 = jnp.full((16,), 1.0, jnp.float32)

  def load_descs(t, slot):
    b = wid + NW * t
    return (
        pltpu.make_async_copy(nodes_hbm.at[pl.ds(b * BIG, BIG)],
                              rows_v.at[slot], sem_r[slot]),
        pltpu.make_async_copy(ids_hbm.at[pl.ds(b, 1)],
                              ids_v.at[slot], sem_i[slot]),
    )

  for t in range(T_STEPS):
    slot = t % 2
    if t == 0:
      @pl.when(wid + NW * t < NBIG)
      def _(t=t, slot=slot):
        for d in load_descs(t, slot):
          d.start()
    if t + 1 < T_STEPS:
      @pl.when(wid + NW * (t + 1) < NBIG)
      def _(t=t, slot=slot):
        for d in load_descs(t + 1, 1 - slot):
          d.start()

    @pl.when(wid + NW * t < NBIG)
    def _(t=t, slot=slot):
      for d in load_descs(t, slot):
        d.wait()
      descs = []
      for j in range(MULTI):
        descs.append(pltpu.async_copy(
            rows_v.at[slot, pl.ds(j * CH, CH)],
            acc_sh.at[ids_v.at[slot, 0, j]], sem_r[slot], add=True))
      for j in range(MULTI):
        for k in range(CH // 16):
          idv = ids_v[slot, 0, j, pl.ds(k * 16, 16)]
          plsc.addupdate_scatter(cnt_v, [idv, lane], one16)
      for d in descs:
        d.wait()

  plsc.subcore_barrier()

  @pl.when(s == 0)
  def _():
    pltpu.sync_copy(acc_sh, sums_hbm.at[c])
  pltpu.sync_copy(cnt_v, counts_hbm.at[wid])


@jax.jit
def _sc_segment_sums(nodes, ids2d, zeros, zcnt):
  mesh = plsc.VectorSubcoreMesh(core_axis_name="c", subcore_axis_name="s")
  cp = pltpu.CompilerParams()
  if "needs_layout_passes" in pltpu.CompilerParams.__dataclass_fields__:
    cp = dataclasses.replace(cp, needs_layout_passes=False)
  kern = pl.kernel(
      _sc_body,
      out_type=(
          jax.ShapeDtypeStruct((NC, N_SEG, D), jnp.float32),
          jax.ShapeDtypeStruct((NW, N_SEG, CNT_W), jnp.float32),
      ),
      mesh=mesh,
      scratch_types=[
          pltpu.VMEM((2, BIG, D), jnp.float32),
          pltpu.VMEM((2, 1, MULTI, CH), jnp.int32),
          pltpu.VMEM((N_SEG, CNT_W), jnp.float32),
          pltpu.VMEM_SHARED((N_SEG, D), jnp.float32),
          pltpu.SemaphoreType.DMA,
          pltpu.SemaphoreType.DMA,
          pltpu.SemaphoreType.DMA,
          pltpu.SemaphoreType.DMA,
      ],
      compiler_params=cp,
  )
  return kern(nodes, ids2d, zeros, zcnt)


def _tc_partial_body(x_ref, ids_ref, sums_ref, cnts_ref):
  i = pl.program_id(0)
  oh_t = (lax.broadcasted_iota(jnp.int32, (N_SEG, 1), 0)
          == ids_ref[0]).astype(jnp.bfloat16)
  x = x_ref[...].astype(jnp.bfloat16)
  part = lax.dot_general(oh_t, x, (((1,), (0,)), ((), ())),
                         preferred_element_type=jnp.float32)
  ones8 = jnp.ones((RB, 8), jnp.bfloat16)
  pcnt = lax.dot_general(oh_t, ones8, (((1,), (0,)), ((), ())),
                         preferred_element_type=jnp.float32)

  @pl.when(i == 0)
  def _():
    sums_ref[...] = jnp.zeros_like(sums_ref)
    cnts_ref[...] = jnp.zeros_like(cnts_ref)

  sums_ref[...] += part
  cnts_ref[...] += pcnt


@jax.jit
def _tc_partial(nodes, ids_tc):
  return pl.pallas_call(
      _tc_partial_body,
      grid=(TC_BLOCKS,),
      in_specs=[
          pl.BlockSpec((RB, D), lambda i: (TC_OFF + i, 0)),
          pl.BlockSpec((1, 1, RB), lambda i: (TC_OFF + i, 0, 0)),
      ],
      out_specs=[
          pl.BlockSpec((N_SEG, D), lambda i: (0, 0)),
          pl.BlockSpec((N_SEG, 8), lambda i: (0, 0)),
      ],
      out_shape=[
          jax.ShapeDtypeStruct((N_SEG, D), jnp.float32),
          jax.ShapeDtypeStruct((N_SEG, 8), jnp.float32),
      ],
  )(nodes, ids_tc)


def _tc_body(sums_ref, counts_ref, tcs_ref, tcc_ref, ctx_ref, w_ref, b_ref,
             out_ref):
  sums = sums_ref[0] + sums_ref[1] + tcs_ref[...]
  cnt = (counts_ref[...].sum(axis=0).sum(axis=-1)[:, None]
         + tcc_ref[:, 0:1])
  pooled = sums / jnp.maximum(cnt, 1.0)
  w_ctx = w_ref[0:D, :]
  w_pool = w_ref[D:2 * D, :]
  out = (
      lax.dot_general(ctx_ref[...], w_ctx, (((1,), (0,)), ((), ())),
                      preferred_element_type=jnp.float32)
      + lax.dot_general(pooled, w_pool, (((1,), (0,)), ((), ())),
                        preferred_element_type=jnp.float32)
      + b_ref[...]
  )
  out_ref[...] = jnp.maximum(out, 0.0)


@jax.jit
def _tc_finish(sums, counts, tc_sums, tc_cnts, context_state, w, b2d):
  return pl.pallas_call(
      _tc_body,
      out_shape=jax.ShapeDtypeStruct((N_SEG, D), jnp.float32),
  )(sums, counts, tc_sums, tc_cnts, context_state, w, b2d)


def kernel(node_states, segment_ids, context_state, W, b):
  ids32 = segment_ids.astype(jnp.int32)
  ids2d = ids32[:SC_ROWS].reshape(NBIG, MULTI, CH)
  ids_tc = ids32.reshape(N_NODES // RB, 1, RB)
  zeros = jnp.zeros((N_SEG, D), jnp.float32)
  zcnt = jnp.zeros((N_SEG, CNT_W), jnp.float32)
  sums, counts = _sc_segment_sums(node_states, ids2d, zeros, zcnt)
  tc_sums, tc_cnts = _tc_partial(node_states, ids_tc)
  return _tc_finish(sums, counts, tc_sums, tc_cnts, context_state, W,
                    b.reshape(1, D))


# speedup vs baseline: 1.0283x; 1.0283x over previous
import dataclasses

import jax
import jax.numpy as jnp
from jax import lax
from jax.experimental import pallas as pl
from jax.experimental.pallas import tpu as pltpu
from jax.experimental.pallas import tpu_sc as plsc

N_NODES = 100000
N_SEG = 256
D = 128
CH = 80
NCHUNKS = N_NODES // CH
NC = 2
NS = 16
NW = NC * NS
CNT_W = 16
MULTI = 4
BIG = MULTI * CH
NBIG = 190
SC_ROWS = NBIG * BIG
T_STEPS = -(-NBIG // NW)
NROWS_IDS = -(-NCHUNKS // MULTI)
T2_STEPS = -(-(NROWS_IDS - NBIG) // NW)
PAD_ID = N_SEG
RB = 800
TC_BLOCKS = (N_NODES - SC_ROWS) // RB
TC_OFF = SC_ROWS // RB


def _sc_body(nodes_hbm, ids_hbm, zeros_hbm, zcnt_hbm, sums_hbm,
             counts_hbm, rows_v, ids_v, cnt_v, acc_sh,
             sem_r0, sem_r1, sem_i0, sem_i1):
  c = lax.axis_index("c")
  s = lax.axis_index("s")
  wid = c * NS + s
  sem_r = (sem_r0, sem_r1)
  sem_i = (sem_i0, sem_i1)

  pltpu.sync_copy(zcnt_hbm, cnt_v)

  @pl.when(s == 0)
  def _():
    pltpu.sync_copy(zeros_hbm, acc_sh)

  plsc.subcore_barrier()

  lane = lax.iota(jnp.int32, 16)
  one16 = jnp.full((16,), 1.0, jnp.float32)

  def load_descs(t, slot):
    b = wid + NW * t
    return (
        pltpu.make_async_copy(nodes_hbm.at[pl.ds(b * BIG, BIG)],
                              rows_v.at[slot], sem_r[slot]),
        pltpu.make_async_copy(ids_hbm.at[pl.ds(b, 1)],
                              ids_v.at[slot], sem_i[slot]),
    )

  for t in range(T_STEPS):
    slot = t % 2
    if t == 0:
      @pl.when(wid + NW * t < NBIG)
      def _(t=t, slot=slot):
        for d in load_descs(t, slot):
          d.start()
    if t + 1 < T_STEPS:
      @pl.when(wid + NW * (t + 1) < NBIG)
      def _(t=t, slot=slot):
        for d in load_descs(t + 1, 1 - slot):
          d.start()

    @pl.when(wid + NW * t < NBIG)
    def _(t=t, slot=slot):
      for d in load_descs(t, slot):
        d.wait()
      descs = []
      for j in range(MULTI):
        descs.append(pltpu.async_copy(
            rows_v.at[slot, pl.ds(j * CH, CH)],
            acc_sh.at[ids_v.at[slot, 0, j]], sem_r[slot], add=True))
      for j in range(MULTI):
        for k in range(CH // 16):
          idv = ids_v[slot, 0, j, pl.ds(k * 16, 16)]
          plsc.addupdate_scatter(cnt_v, [idv, lane], one16)
      for d in descs:
        d.wait()

  for t2 in range(T2_STEPS):
    b2 = NBIG + wid + NW * t2

    @pl.when(b2 < NROWS_IDS)
    def _(b2=b2):
      pltpu.sync_copy(ids_hbm.at[pl.ds(b2, 1)], ids_v.at[0])
      for j in range(MULTI):
        for k in range(CH // 16):
          idv = ids_v[0, 0, j, pl.ds(k * 16, 16)]
          plsc.addupdate_scatter(cnt_v, [idv, lane], one16)

  plsc.subcore_barrier()

  @pl.when(s == 0)
  def _():
    pltpu.sync_copy(acc_sh, sums_hbm.at[c])
  pltpu.sync_copy(cnt_v.at[pl.ds(0, N_SEG)], counts_hbm.at[wid])


@jax.jit
def _sc_segment_sums(nodes, ids2d, zeros, zcnt):
  mesh = plsc.VectorSubcoreMesh(core_axis_name="c", subcore_axis_name="s")
  cp = pltpu.CompilerParams()
  if "needs_layout_passes" in pltpu.CompilerParams.__dataclass_fields__:
    cp = dataclasses.replace(cp, needs_layout_passes=False)
  kern = pl.kernel(
      _sc_body,
      out_type=(
          jax.ShapeDtypeStruct((NC, N_SEG, D), jnp.float32),
          jax.ShapeDtypeStruct((NW, N_SEG, CNT_W), jnp.float32),
      ),
      mesh=mesh,
      scratch_types=[
          pltpu.VMEM((2, BIG, D), jnp.float32),
          pltpu.VMEM((2, 1, MULTI, CH), jnp.int32),
          pltpu.VMEM((N_SEG + 1, CNT_W), jnp.float32),
          pltpu.VMEM_SHARED((N_SEG, D), jnp.float32),
          pltpu.SemaphoreType.DMA,
          pltpu.SemaphoreType.DMA,
          pltpu.SemaphoreType.DMA,
          pltpu.SemaphoreType.DMA,
      ],
      compiler_params=cp,
  )
  return kern(nodes, ids2d, zeros, zcnt)


def _tc_partial_body(x_ref, ids_ref, sums_ref):
  i = pl.program_id(0)
  iot = lax.broadcasted_iota(jnp.int32, (N_SEG, 1), 0).astype(jnp.bfloat16)
  oh_t = (iot == ids_ref[0].astype(jnp.bfloat16)).astype(jnp.bfloat16)
  x = x_ref[...].astype(jnp.bfloat16)
  part = lax.dot_general(oh_t, x, (((1,), (0,)), ((), ())),
                         preferred_element_type=jnp.float32)

  @pl.when(i == 0)
  def _():
    sums_ref[...] = jnp.zeros_like(sums_ref)

  sums_ref[...] += part


@jax.jit
def _tc_partial(nodes, ids_tc):
  return pl.pallas_call(
      _tc_partial_body,
      grid=(TC_BLOCKS,),
      in_specs=[
          pl.BlockSpec((RB, D), lambda i: (TC_OFF + i, 0)),
          pl.BlockSpec((1, 1, RB), lambda i: (TC_OFF + i, 0, 0)),
      ],
      out_specs=pl.BlockSpec((N_SEG, D), lambda i: (0, 0)),
      out_shape=jax.ShapeDtypeStruct((N_SEG, D), jnp.float32),
  )(nodes, ids_tc)


def _tc_body(sums_ref, counts_ref, tcs_ref, ctx_ref, w_ref, b_ref,
             out_ref):
  sums = sums_ref[0] + sums_ref[1] + tcs_ref[...]
  cnt = counts_ref[...].sum(axis=0).sum(axis=-1)[:, None]
  pooled = sums / jnp.maximum(cnt, 1.0)
  w_ctx = w_ref[0:D, :]
  w_pool = w_ref[D:2 * D, :]
  out = (
      lax.dot_general(ctx_ref[...], w_ctx, (((1,), (0,)), ((), ())),
                      preferred_element_type=jnp.float32)
      + lax.dot_general(pooled, w_pool, (((1,), (0,)), ((), ())),
                        preferred_element_type=jnp.float32)
      + b_ref[...]
  )
  out_ref[...] = jnp.maximum(out, 0.0)


@jax.jit
def _tc_finish(sums, counts, tc_sums, context_state, w, b2d):
  return pl.pallas_call(
      _tc_body,
      out_shape=jax.ShapeDtypeStruct((N_SEG, D), jnp.float32),
  )(sums, counts, tc_sums, context_state, w, b2d)


def kernel(node_states, segment_ids, context_state, W, b):
  ids32 = segment_ids.astype(jnp.int32)
  pad = NROWS_IDS * MULTI * CH - N_NODES
  ids3d = jnp.concatenate(
      [ids32, jnp.full((pad,), PAD_ID, jnp.int32)]
  ).reshape(NROWS_IDS, MULTI, CH)
  ids_tc = ids32.reshape(N_NODES // RB, 1, RB)
  zeros = jnp.zeros((N_SEG, D), jnp.float32)
  zcnt = jnp.zeros((N_SEG + 1, CNT_W), jnp.float32)
  sums, counts = _sc_segment_sums(node_states, ids3d, zeros, zcnt)
  tc_sums = _tc_partial(node_states, ids_tc)
  return _tc_finish(sums, counts, tc_sums, context_state, W,
                    b.reshape(1, D))


# speedup vs baseline: 1.1587x; 1.1268x over previous
import dataclasses

import jax
import jax.numpy as jnp
from jax import lax
from jax.experimental import pallas as pl
from jax.experimental.pallas import tpu as pltpu
from jax.experimental.pallas import tpu_sc as plsc

N_NODES = 100000
N_SEG = 256
D = 128
CH = 80
NCHUNKS = N_NODES // CH
NC = 2
NS = 16
NW = NC * NS
CNT_W = 16
MULTI = 4
BIG = MULTI * CH
NBIG = 175
SC_ROWS = NBIG * BIG
T_STEPS = -(-NBIG // NW)
NROWS_IDS = -(-NCHUNKS // MULTI)
T2_STEPS = -(-(NROWS_IDS - NBIG) // NW)
PAD_ID = N_SEG
RB = 2000
TC_BLOCKS = (N_NODES - SC_ROWS) // RB
TC_OFF = SC_ROWS // RB


def _sc_body(nodes_hbm, ids_hbm, zeros_hbm, zcnt_hbm, sums_hbm,
             counts_hbm, rows_v, ids_v, cnt_v, acc_sh,
             sem_r0, sem_r1, sem_i0, sem_i1):
  c = lax.axis_index("c")
  s = lax.axis_index("s")
  wid = c * NS + s
  sem_r = (sem_r0, sem_r1)
  sem_i = (sem_i0, sem_i1)

  pltpu.sync_copy(zcnt_hbm, cnt_v)

  @pl.when(s == 0)
  def _():
    pltpu.sync_copy(zeros_hbm, acc_sh)

  plsc.subcore_barrier()

  lane = lax.iota(jnp.int32, 16)
  one16 = jnp.full((16,), 1.0, jnp.float32)

  def load_descs(t, slot):
    b = wid + NW * t
    return (
        pltpu.make_async_copy(nodes_hbm.at[pl.ds(b * BIG, BIG)],
                              rows_v.at[slot], sem_r[slot]),
        pltpu.make_async_copy(ids_hbm.at[pl.ds(b, 1)],
                              ids_v.at[slot], sem_i[slot]),
    )

  for t in range(T_STEPS):
    slot = t % 2
    if t == 0:
      @pl.when(wid + NW * t < NBIG)
      def _(t=t, slot=slot):
        for d in load_descs(t, slot):
          d.start()
    if t + 1 < T_STEPS:
      @pl.when(wid + NW * (t + 1) < NBIG)
      def _(t=t, slot=slot):
        for d in load_descs(t + 1, 1 - slot):
          d.start()

    @pl.when(wid + NW * t < NBIG)
    def _(t=t, slot=slot):
      for d in load_descs(t, slot):
        d.wait()
      descs = []
      for j in range(MULTI):
        descs.append(pltpu.async_copy(
            rows_v.at[slot, pl.ds(j * CH, CH)],
            acc_sh.at[ids_v.at[slot, 0, j]], sem_r[slot], add=True))
      for j in range(MULTI):
        for k in range(CH // 16):
          idv = ids_v[slot, 0, j, pl.ds(k * 16, 16)]
          plsc.addupdate_scatter(cnt_v, [idv, lane], one16)
      for d in descs:
        d.wait()

  for t2 in range(T2_STEPS):
    b2 = NBIG + wid + NW * t2

    @pl.when(b2 < NROWS_IDS)
    def _(b2=b2):
      pltpu.sync_copy(ids_hbm.at[pl.ds(b2, 1)], ids_v.at[0])
      for j in range(MULTI):
        for k in range(CH // 16):
          idv = ids_v[0, 0, j, pl.ds(k * 16, 16)]
          plsc.addupdate_scatter(cnt_v, [idv, lane], one16)

  plsc.subcore_barrier()

  @pl.when(s == 0)
  def _():
    pltpu.sync_copy(acc_sh, sums_hbm.at[c])
  pltpu.sync_copy(cnt_v.at[pl.ds(0, N_SEG)], counts_hbm.at[wid])


@jax.jit
def _sc_segment_sums(nodes, ids2d, zeros, zcnt):
  mesh = plsc.VectorSubcoreMesh(core_axis_name="c", subcore_axis_name="s")
  cp = pltpu.CompilerParams()
  if "needs_layout_passes" in pltpu.CompilerParams.__dataclass_fields__:
    cp = dataclasses.replace(cp, needs_layout_passes=False)
  kern = pl.kernel(
      _sc_body,
      out_type=(
          jax.ShapeDtypeStruct((NC, N_SEG, D), jnp.float32),
          jax.ShapeDtypeStruct((NW, N_SEG, CNT_W), jnp.float32),
      ),
      mesh=mesh,
      scratch_types=[
          pltpu.VMEM((2, BIG, D), jnp.float32),
          pltpu.VMEM((2, 1, MULTI, CH), jnp.int32),
          pltpu.VMEM((N_SEG + 1, CNT_W), jnp.float32),
          pltpu.VMEM_SHARED((N_SEG, D), jnp.float32),
          pltpu.SemaphoreType.DMA,
          pltpu.SemaphoreType.DMA,
          pltpu.SemaphoreType.DMA,
          pltpu.SemaphoreType.DMA,
      ],
      compiler_params=cp,
  )
  return kern(nodes, ids2d, zeros, zcnt)


def _tc_partial_body(x_ref, ids_ref, sums_ref):
  i = pl.program_id(0)
  iot = lax.broadcasted_iota(jnp.int32, (N_SEG, 1), 0).astype(jnp.bfloat16)
  oh_t = (iot == ids_ref[0].astype(jnp.bfloat16)).astype(jnp.bfloat16)
  x = x_ref[...].astype(jnp.bfloat16)
  part = lax.dot_general(oh_t, x, (((1,), (0,)), ((), ())),
                         preferred_element_type=jnp.float32)

  @pl.when(i == 0)
  def _():
    sums_ref[...] = jnp.zeros_like(sums_ref)

  sums_ref[...] += part


@jax.jit
def _tc_partial(nodes, ids_tc):
  return pl.pallas_call(
      _tc_partial_body,
      grid=(TC_BLOCKS,),
      in_specs=[
          pl.BlockSpec((RB, D), lambda i: (TC_OFF + i, 0)),
          pl.BlockSpec((1, 1, RB), lambda i: (TC_OFF + i, 0, 0)),
      ],
      out_specs=pl.BlockSpec((N_SEG, D), lambda i: (0, 0)),
      out_shape=jax.ShapeDtypeStruct((N_SEG, D), jnp.float32),
  )(nodes, ids_tc)


def _tc_body(sums_ref, counts_ref, tcs_ref, ctx_ref, w_ref, b_ref,
             out_ref):
  sums = sums_ref[0] + sums_ref[1] + tcs_ref[...]
  cnt = counts_ref[...].sum(axis=0).sum(axis=-1)[:, None]
  pooled = sums / jnp.maximum(cnt, 1.0)
  w_ctx = w_ref[0:D, :]
  w_pool = w_ref[D:2 * D, :]
  out = (
      lax.dot_general(ctx_ref[...], w_ctx, (((1,), (0,)), ((), ())),
                      preferred_element_type=jnp.float32)
      + lax.dot_general(pooled, w_pool, (((1,), (0,)), ((), ())),
                        preferred_element_type=jnp.float32)
      + b_ref[...]
  )
  out_ref[...] = jnp.maximum(out, 0.0)


@jax.jit
def _tc_finish(sums, counts, tc_sums, context_state, w, b2d):
  return pl.pallas_call(
      _tc_body,
      out_shape=jax.ShapeDtypeStruct((N_SEG, D), jnp.float32),
  )(sums, counts, tc_sums, context_state, w, b2d)


def kernel(node_states, segment_ids, context_state, W, b):
  ids32 = segment_ids.astype(jnp.int32)
  pad = NROWS_IDS * MULTI * CH - N_NODES
  ids3d = jnp.concatenate(
      [ids32, jnp.full((pad,), PAD_ID, jnp.int32)]
  ).reshape(NROWS_IDS, MULTI, CH)
  ids_tc = ids32.reshape(N_NODES // RB, 1, RB)
  zeros = jnp.zeros((N_SEG, D), jnp.float32)
  zcnt = jnp.zeros((N_SEG + 1, CNT_W), jnp.float32)
  sums, counts = _sc_segment_sums(node_states, ids3d, zeros, zcnt)
  tc_sums = _tc_partial(node_states, ids_tc)
  return _tc_finish(sums, counts, tc_sums, context_state, W,
                    b.reshape(1, D))


# speedup vs baseline: 1.2349x; 1.0657x over previous
import dataclasses

import jax
import jax.numpy as jnp
from jax import lax
from jax.experimental import pallas as pl
from jax.experimental.pallas import tpu as pltpu
from jax.experimental.pallas import tpu_sc as plsc

N_NODES = 100000
N_SEG = 256
D = 128
CH = 80
NCHUNKS = N_NODES // CH
NC = 2
NS = 16
NW = NC * NS
CNT_W = 16
MULTI = 4
BIG = MULTI * CH
NBIG = 175
SC_ROWS = NBIG * BIG
T_STEPS = -(-NBIG // NW)
RB = 4000
TC_BLOCKS = (N_NODES - SC_ROWS) // RB
TC_OFF = SC_ROWS // RB


def _sc_body(nodes_hbm, ids_hbm, zeros_hbm, zcnt_hbm, sums_hbm,
             counts_hbm, rows_v, ids_v, cnt_v, acc_sh,
             sem_r0, sem_r1, sem_i0, sem_i1):
  c = lax.axis_index("c")
  s = lax.axis_index("s")
  wid = c * NS + s
  sem_r = (sem_r0, sem_r1)
  sem_i = (sem_i0, sem_i1)

  pltpu.sync_copy(zcnt_hbm, cnt_v)

  @pl.when(s == 0)
  def _():
    pltpu.sync_copy(zeros_hbm, acc_sh)

  plsc.subcore_barrier()

  lane = lax.iota(jnp.int32, 16)
  one16 = jnp.full((16,), 1.0, jnp.float32)

  def load_descs(t, slot):
    b = wid + NW * t
    return (
        pltpu.make_async_copy(nodes_hbm.at[pl.ds(b * BIG, BIG)],
                              rows_v.at[slot], sem_r[slot]),
        pltpu.make_async_copy(ids_hbm.at[pl.ds(b, 1)],
                              ids_v.at[slot], sem_i[slot]),
    )

  for t in range(T_STEPS):
    slot = t % 2
    if t == 0:
      @pl.when(wid + NW * t < NBIG)
      def _(t=t, slot=slot):
        for d in load_descs(t, slot):
          d.start()
    if t + 1 < T_STEPS:
      @pl.when(wid + NW * (t + 1) < NBIG)
      def _(t=t, slot=slot):
        for d in load_descs(t + 1, 1 - slot):
          d.start()

    @pl.when(wid + NW * t < NBIG)
    def _(t=t, slot=slot):
      for d in load_descs(t, slot):
        d.wait()
      descs = []
      for j in range(MULTI):
        descs.append(pltpu.async_copy(
            rows_v.at[slot, pl.ds(j * CH, CH)],
            acc_sh.at[ids_v.at[slot, 0, j]], sem_r[slot], add=True))
      for j in range(MULTI):
        for k in range(CH // 16):
          idv = ids_v[slot, 0, j, pl.ds(k * 16, 16)]
          plsc.addupdate_scatter(cnt_v, [idv, lane], one16)
      for d in descs:
        d.wait()

  plsc.subcore_barrier()

  @pl.when(s == 0)
  def _():
    pltpu.sync_copy(acc_sh, sums_hbm.at[c])
  pltpu.sync_copy(cnt_v, counts_hbm.at[wid])


@jax.jit
def _sc_segment_sums(nodes, ids2d, zeros, zcnt):
  mesh = plsc.VectorSubcoreMesh(core_axis_name="c", subcore_axis_name="s")
  cp = pltpu.CompilerParams()
  if "needs_layout_passes" in pltpu.CompilerParams.__dataclass_fields__:
    cp = dataclasses.replace(cp, needs_layout_passes=False)
  kern = pl.kernel(
      _sc_body,
      out_type=(
          jax.ShapeDtypeStruct((NC, N_SEG, D), jnp.float32),
          jax.ShapeDtypeStruct((NW, N_SEG, CNT_W), jnp.float32),
      ),
      mesh=mesh,
      scratch_types=[
          pltpu.VMEM((2, BIG, D), jnp.float32),
          pltpu.VMEM((2, 1, MULTI, CH), jnp.int32),
          pltpu.VMEM((N_SEG, CNT_W), jnp.float32),
          pltpu.VMEM_SHARED((N_SEG, D), jnp.float32),
          pltpu.SemaphoreType.DMA,
          pltpu.SemaphoreType.DMA,
          pltpu.SemaphoreType.DMA,
          pltpu.SemaphoreType.DMA,
      ],
      compiler_params=cp,
  )
  return kern(nodes, ids2d, zeros, zcnt)


def _tc_partial_body(x_ref, ids_ref, sums_ref, cnts_ref):
  i = pl.program_id(0)
  iot = lax.broadcasted_iota(jnp.int32, (N_SEG, 1), 0).astype(jnp.bfloat16)
  oh_t = (iot == ids_ref[0].astype(jnp.bfloat16)).astype(jnp.bfloat16)
  x = x_ref[...].astype(jnp.bfloat16)
  part = lax.dot_general(oh_t, x, (((1,), (0,)), ((), ())),
                         preferred_element_type=jnp.float32)
  pcnt = jnp.sum(oh_t.astype(jnp.float32), axis=1, keepdims=True)

  @pl.when(i == 0)
  def _():
    sums_ref[...] = jnp.zeros_like(sums_ref)
    cnts_ref[...] = jnp.zeros_like(cnts_ref)

  sums_ref[...] += part
  cnts_ref[...] += jnp.broadcast_to(pcnt, (N_SEG, 8))


@jax.jit
def _tc_partial(nodes, ids_tc):
  return pl.pallas_call(
      _tc_partial_body,
      grid=(TC_BLOCKS,),
      in_specs=[
          pl.BlockSpec((RB, D), lambda i: (TC_OFF + i, 0)),
          pl.BlockSpec((1, 1, RB), lambda i: (TC_OFF + i, 0, 0)),
      ],
      out_specs=[
          pl.BlockSpec((N_SEG, D), lambda i: (0, 0)),
          pl.BlockSpec((N_SEG, 8), lambda i: (0, 0)),
      ],
      out_shape=[
          jax.ShapeDtypeStruct((N_SEG, D), jnp.float32),
          jax.ShapeDtypeStruct((N_SEG, 8), jnp.float32),
      ],
  )(nodes, ids_tc)


def _tc_body(sums_ref, counts_ref, tcs_ref, tcc_ref, ctx_ref, w_ref, b_ref,
             out_ref):
  sums = sums_ref[0] + sums_ref[1] + tcs_ref[...]
  cnt = (counts_ref[...].sum(axis=0).sum(axis=-1)[:, None]
         + tcc_ref[:, 0:1])
  pooled = sums / jnp.maximum(cnt, 1.0)
  w_ctx = w_ref[0:D, :]
  w_pool = w_ref[D:2 * D, :]
  out = (
      lax.dot_general(ctx_ref[...], w_ctx, (((1,), (0,)), ((), ())),
                      preferred_element_type=jnp.float32)
      + lax.dot_general(pooled, w_pool, (((1,), (0,)), ((), ())),
                        preferred_element_type=jnp.float32)
      + b_ref[...]
  )
  out_ref[...] = jnp.maximum(out, 0.0)


@jax.jit
def _tc_finish(sums, counts, tc_sums, tc_cnts, context_state, w, b2d):
  return pl.pallas_call(
      _tc_body,
      out_shape=jax.ShapeDtypeStruct((N_SEG, D), jnp.float32),
  )(sums, counts, tc_sums, tc_cnts, context_state, w, b2d)


def kernel(node_states, segment_ids, context_state, W, b):
  ids32 = segment_ids.astype(jnp.int32)
  ids3d = ids32[:SC_ROWS].reshape(NBIG, MULTI, CH)
  ids_tc = ids32.reshape(N_NODES // RB, 1, RB)
  zeros = jnp.zeros((N_SEG, D), jnp.float32)
  zcnt = jnp.zeros((N_SEG, CNT_W), jnp.float32)
  sums, counts = _sc_segment_sums(node_states, ids3d, zeros, zcnt)
  tc_sums, tc_cnts = _tc_partial(node_states, ids_tc)
  return _tc_finish(sums, counts, tc_sums, tc_cnts, context_state, W,
                    b.reshape(1, D))


# speedup vs baseline: 1.2631x; 1.0228x over previous
import dataclasses

import jax
import jax.numpy as jnp
from jax import lax
from jax.experimental import pallas as pl
from jax.experimental.pallas import tpu as pltpu
from jax.experimental.pallas import tpu_sc as plsc

N_NODES = 100000
N_SEG = 256
D = 128
CH = 80
NCHUNKS = N_NODES // CH
NC = 2
NS = 16
NW = NC * NS
CNT_W = 16
MULTI = 4
BIG = MULTI * CH
NBIG = 150
SC_ROWS = NBIG * BIG
T_STEPS = -(-NBIG // NW)
RB = 4000
TC_BLOCKS = (N_NODES - SC_ROWS) // RB
TC_OFF = SC_ROWS // RB


def _sc_body(nodes_hbm, ids_hbm, zeros_hbm, zcnt_hbm, sums_hbm,
             counts_hbm, rows_v, ids_v, cnt_v, acc_sh,
             sem_r0, sem_r1, sem_i0, sem_i1):
  c = lax.axis_index("c")
  s = lax.axis_index("s")
  wid = c * NS + s
  sem_r = (sem_r0, sem_r1)
  sem_i = (sem_i0, sem_i1)

  pltpu.sync_copy(zcnt_hbm, cnt_v)

  @pl.when(s == 0)
  def _():
    pltpu.sync_copy(zeros_hbm, acc_sh)

  plsc.subcore_barrier()

  lane = lax.iota(jnp.int32, 16)
  one16 = jnp.full((16,), 1.0, jnp.float32)

  def load_descs(t, slot):
    b = wid + NW * t
    return (
        pltpu.make_async_copy(nodes_hbm.at[pl.ds(b * BIG, BIG)],
                              rows_v.at[slot], sem_r[slot]),
        pltpu.make_async_copy(ids_hbm.at[pl.ds(b, 1)],
                              ids_v.at[slot], sem_i[slot]),
    )

  for t in range(T_STEPS):
    slot = t % 2
    if t == 0:
      @pl.when(wid + NW * t < NBIG)
      def _(t=t, slot=slot):
        for d in load_descs(t, slot):
          d.start()
    if t + 1 < T_STEPS:
      @pl.when(wid + NW * (t + 1) < NBIG)
      def _(t=t, slot=slot):
        for d in load_descs(t + 1, 1 - slot):
          d.start()

    @pl.when(wid + NW * t < NBIG)
    def _(t=t, slot=slot):
      for d in load_descs(t, slot):
        d.wait()
      descs = []
      for j in range(MULTI):
        descs.append(pltpu.async_copy(
            rows_v.at[slot, pl.ds(j * CH, CH)],
            acc_sh.at[ids_v.at[slot, 0, j]], sem_r[slot], add=True))
      for j in range(MULTI):
        for k in range(CH // 16):
          idv = ids_v[slot, 0, j, pl.ds(k * 16, 16)]
          plsc.addupdate_scatter(cnt_v, [idv, lane], one16)
      for d in descs:
        d.wait()

  plsc.subcore_barrier()

  @pl.when(s == 0)
  def _():
    pltpu.sync_copy(acc_sh, sums_hbm.at[c])
  pltpu.sync_copy(cnt_v, counts_hbm.at[wid])


@jax.jit
def _sc_segment_sums(nodes, ids2d, zeros, zcnt):
  mesh = plsc.VectorSubcoreMesh(core_axis_name="c", subcore_axis_name="s")
  cp = pltpu.CompilerParams()
  if "needs_layout_passes" in pltpu.CompilerParams.__dataclass_fields__:
    cp = dataclasses.replace(cp, needs_layout_passes=False)
  kern = pl.kernel(
      _sc_body,
      out_type=(
          jax.ShapeDtypeStruct((NC, N_SEG, D), jnp.float32),
          jax.ShapeDtypeStruct((NW, N_SEG, CNT_W), jnp.float32),
      ),
      mesh=mesh,
      scratch_types=[
          pltpu.VMEM((2, BIG, D), jnp.float32),
          pltpu.VMEM((2, 1, MULTI, CH), jnp.int32),
          pltpu.VMEM((N_SEG, CNT_W), jnp.float32),
          pltpu.VMEM_SHARED((N_SEG, D), jnp.float32),
          pltpu.SemaphoreType.DMA,
          pltpu.SemaphoreType.DMA,
          pltpu.SemaphoreType.DMA,
          pltpu.SemaphoreType.DMA,
      ],
      compiler_params=cp,
  )
  return kern(nodes, ids2d, zeros, zcnt)


def _tc_partial_body(x_ref, ids_ref, sums_ref, cnts_ref):
  i = pl.program_id(0)
  iot = lax.broadcasted_iota(jnp.int32, (N_SEG, 1), 0).astype(jnp.bfloat16)
  oh_t = (iot == ids_ref[0].astype(jnp.bfloat16)).astype(jnp.bfloat16)
  x = x_ref[...].astype(jnp.bfloat16)
  part = lax.dot_general(oh_t, x, (((1,), (0,)), ((), ())),
                         preferred_element_type=jnp.float32)
  pcnt = jnp.sum(oh_t.astype(jnp.float32), axis=1, keepdims=True)

  @pl.when(i == 0)
  def _():
    sums_ref[...] = jnp.zeros_like(sums_ref)
    cnts_ref[...] = jnp.zeros_like(cnts_ref)

  sums_ref[...] += part
  cnts_ref[...] += jnp.broadcast_to(pcnt, (N_SEG, 8))


@jax.jit
def _tc_partial(nodes, ids_tc):
  return pl.pallas_call(
      _tc_partial_body,
      grid=(TC_BLOCKS,),
      in_specs=[
          pl.BlockSpec((RB, D), lambda i: (TC_OFF + i, 0)),
          pl.BlockSpec((1, 1, RB), lambda i: (TC_OFF + i, 0, 0)),
      ],
      out_specs=[
          pl.BlockSpec((N_SEG, D), lambda i: (0, 0)),
          pl.BlockSpec((N_SEG, 8), lambda i: (0, 0)),
      ],
      out_shape=[
          jax.ShapeDtypeStruct((N_SEG, D), jnp.float32),
          jax.ShapeDtypeStruct((N_SEG, 8), jnp.float32),
      ],
  )(nodes, ids_tc)


def _tc_body(sums_ref, counts_ref, tcs_ref, tcc_ref, ctx_ref, w_ref, b_ref,
             out_ref):
  sums = sums_ref[0] + sums_ref[1] + tcs_ref[...]
  cnt = (counts_ref[...].sum(axis=0).sum(axis=-1)[:, None]
         + tcc_ref[:, 0:1])
  pooled = sums / jnp.maximum(cnt, 1.0)
  w_ctx = w_ref[0:D, :]
  w_pool = w_ref[D:2 * D, :]
  out = (
      lax.dot_general(ctx_ref[...], w_ctx, (((1,), (0,)), ((), ())),
                      preferred_element_type=jnp.float32)
      + lax.dot_general(pooled, w_pool, (((1,), (0,)), ((), ())),
                        preferred_element_type=jnp.float32)
      + b_ref[...]
  )
  out_ref[...] = jnp.maximum(out, 0.0)


@jax.jit
def _tc_finish(sums, counts, tc_sums, tc_cnts, context_state, w, b2d):
  return pl.pallas_call(
      _tc_body,
      out_shape=jax.ShapeDtypeStruct((N_SEG, D), jnp.float32),
  )(sums, counts, tc_sums, tc_cnts, context_state, w, b2d)


def kernel(node_states, segment_ids, context_state, W, b):
  ids32 = segment_ids.astype(jnp.int32)
  ids3d = ids32[:SC_ROWS].reshape(NBIG, MULTI, CH)
  ids_tc = ids32.reshape(N_NODES // RB, 1, RB)
  zeros = jnp.zeros((N_SEG, D), jnp.float32)
  zcnt = jnp.zeros((N_SEG, CNT_W), jnp.float32)
  sums, counts = _sc_segment_sums(node_states, ids3d, zeros, zcnt)
  tc_sums, tc_cnts = _tc_partial(node_states, ids_tc)
  return _tc_finish(sums, counts, tc_sums, tc_cnts, context_state, W,
                    b.reshape(1, D))
